# Initial kernel scaffold; baseline (speedup 1.0000x reference)
import jax
import jax.numpy as jnp
from jax.experimental import pallas as pl


def kernel(h_i, edge_index, node_params, edge_params, W1, b1, W2, b2, W3, b3):
    raise NotImplementedError("write your pallas kernel here")


# speedup vs baseline: 8.1299x; 8.1299x over previous
import functools
import math

import jax
import jax.numpy as jnp
from jax import lax
from jax.experimental import pallas as pl
from jax.experimental.pallas import tpu as pltpu

D = 8
FLAT = D * D
F = 80
HID = 128


def _mlp_body(ps_ref, pr_ref, ex_ref, w1s_ref, w1r_ref, w1e_ref, b1_ref,
              w2_ref, b2_ref, w3_ref, b3_ref, out_ref):
    ps = ps_ref[...]
    pr = pr_ref[...]
    ex = ex_ref[...]
    x = (jnp.dot(ps, w1s_ref[...], preferred_element_type=jnp.float32)
         + jnp.dot(pr, w1r_ref[...], preferred_element_type=jnp.float32)
         + jnp.dot(ex, w1e_ref[...], preferred_element_type=jnp.float32)
         + b1_ref[...])
    x = x * jax.nn.sigmoid(x)
    x = jnp.dot(x, w2_ref[...], preferred_element_type=jnp.float32) + b2_ref[...]
    x = x * jax.nn.sigmoid(x)
    m = jnp.dot(x, w3_ref[...], preferred_element_type=jnp.float32) + b3_ref[...]
    eb = m.shape[0]
    ones = jnp.ones((eb, 1), jnp.float32)
    zeros = jnp.zeros((eb, F - FLAT - 1), jnp.float32)
    out_ref[...] = jnp.concatenate([m, ones, zeros], axis=1)


def _run_mlp(gath, ex, w1s, w1r, w1e, b1, w2, b2, w3, b3, num_edges):
    eb = 2000 if num_edges % 2000 == 0 else num_edges
    grid = num_edges // eb
    off = num_edges // eb
    wspec = lambda shape: pl.BlockSpec(shape, lambda e: (0, 0))
    return pl.pallas_call(
        _mlp_body,
        grid=(grid,),
        in_specs=[
            pl.BlockSpec((eb, F), lambda e: (e, 0)),
            pl.BlockSpec((eb, F), lambda e: (e + off, 0)),
            pl.BlockSpec((eb, 8), lambda e: (e, 0)),
            wspec((F, HID)), wspec((F, HID)), wspec((8, HID)), wspec((1, HID)),
            wspec((HID, HID)), wspec((1, HID)),
            wspec((HID, FLAT)), wspec((1, FLAT)),
        ],
        out_specs=pl.BlockSpec((eb, F), lambda e: (e, 0)),
        out_shape=jax.ShapeDtypeStruct((num_edges, F), jnp.float32),
    )(gath, gath, ex, w1s, w1r, w1e, b1, w2, b2, w3, b3)


def _t64(m):
    rows = [m[j * 8 + i:j * 8 + i + 1, :] for i in range(8) for j in range(8)]
    return jnp.concatenate(rows, axis=0)


def _bmm(a, b):
    outs = []
    for i in range(8):
        acc = a[i * 8:i * 8 + 1, :] * b[0:8, :]
        for k in range(1, 8):
            acc = acc + a[i * 8 + k:i * 8 + k + 1, :] * b[k * 8:k * 8 + 8, :]
        outs.append(acc)
    return jnp.concatenate(outs, axis=0)


def _expm_body(agg_ref, h_ref, out_ref):
    agg = agg_ref[...]
    cnt = jnp.maximum(agg[FLAT:FLAT + 1, :], 1.0)
    mean = agg[0:FLAT, :] / cnt
    abs_sum = jnp.sum(jnp.abs(mean), axis=0, keepdims=True)
    r = lax.broadcasted_iota(jnp.int32, (FLAT, 1), 0)
    eye = ((r // 8) == (r % 8)).astype(jnp.float32)
    m = jnp.where(abs_sum == 0.0, eye, mean)
    skew = 0.5 * (m - _t64(m))
    a = skew * (1.0 / 256.0)
    b = m.shape[1]
    eye_b = jnp.broadcast_to(eye, (FLAT, b))
    term = eye_b
    result = eye_b
    for k in range(1, 13):
        term = _bmm(term, a) * (1.0 / k)
        result = result + term
    for _ in range(8):
        result = _bmm(result, result)
    u = result
    h = h_ref[...]
    out_ref[...] = _bmm(u, _bmm(h, _t64(u)))


def _run_expm(agg_t, h_t, npad):
    nb = 2048 if npad % 2048 == 0 else npad
    grid = npad // nb
    return pl.pallas_call(
        _expm_body,
        grid=(grid,),
        in_specs=[
            pl.BlockSpec((F, nb), lambda i: (0, i)),
            pl.BlockSpec((FLAT, nb), lambda i: (0, i)),
        ],
        out_specs=pl.BlockSpec((FLAT, nb), lambda i: (0, i)),
        out_shape=jax.ShapeDtypeStruct((FLAT, npad), jnp.float32),
    )(agg_t, h_t)


def kernel(h_i, edge_index, node_params, edge_params, W1, b1, W2, b2, W3, b3):
    n = h_i.shape[0]
    num_edges = edge_index.shape[1]
    sender = edge_index[0]
    receiver = edge_index[1]

    h_flat = h_i.reshape(n, FLAT)
    p = jnp.concatenate(
        [h_flat, node_params, edge_params,
         jnp.zeros((n, F - FLAT - 3), jnp.float32)], axis=1)

    idx_all = jnp.concatenate([sender, receiver])
    gath = p[idx_all]

    eps = gath[:num_edges, 66:67]
    epr = gath[num_edges:, 66:67]
    scram = jnp.concatenate([eps, epr], axis=0).reshape(num_edges, 2)
    ex = jnp.concatenate([scram, jnp.zeros((num_edges, 6), jnp.float32)], axis=1)

    zcol = jnp.zeros((F - 66, HID), jnp.float32)
    w1s = jnp.concatenate([W1[0:64], W1[128:130], zcol], axis=0)
    w1r = jnp.concatenate([W1[64:128], W1[130:132], zcol], axis=0)
    w1e = jnp.concatenate([W1[132:134], jnp.zeros((6, HID), jnp.float32)], axis=0)

    m80 = _run_mlp(gath, ex, w1s, w1r, w1e, b1.reshape(1, HID),
                   W2, b2.reshape(1, HID), W3, b3.reshape(1, FLAT), num_edges)

    agg80 = jnp.zeros((n, F), jnp.float32).at[receiver].add(m80)

    nb = 2048
    npad = ((n + nb - 1) // nb) * nb
    agg_t = jnp.pad(agg80, ((0, npad - n), (0, 0))).T
    h_t = jnp.pad(h_flat, ((0, npad - n), (0, 0))).T

    out_t = _run_expm(agg_t, h_t, npad)
    return out_t.T[:n].reshape(n, D, D)


# speedup vs baseline: 14.1185x; 1.7366x over previous
import functools
import math

import jax
import jax.numpy as jnp
from jax import lax
from jax.experimental import pallas as pl
from jax.experimental.pallas import tpu as pltpu
from jax.experimental.pallas import tpu_sc as plsc

D = 8
FLAT = D * D
F = 80
HID = 128

SC_NC = 2
SC_NS = 16
SC_NW = SC_NC * SC_NS


G_K = 8
G_ROWS = 128
G_CHUNK = G_K * G_ROWS


def _sc_gather(table, idx, steps_per_tile):
    per_tile = steps_per_tile * G_CHUNK
    b = per_tile * SC_NW
    assert idx.shape[0] == b
    idx2 = idx.reshape(b // G_ROWS, G_ROWS)
    mesh = plsc.VectorSubcoreMesh(core_axis_name="c", subcore_axis_name="s")

    @functools.partial(
        pl.kernel, mesh=mesh,
        out_type=jax.ShapeDtypeStruct((b, F), jnp.float32),
        scratch_types=[
            pltpu.VMEM((G_K, G_ROWS), jnp.int32),
            pltpu.VMEM((G_CHUNK, F), jnp.float32),
            pltpu.SemaphoreType.DMA,
        ],
        compiler_params=pltpu.CompilerParams(use_tc_tiling_on_sc=False),
    )
    def gk(table_hbm, idx_hbm, out_hbm, idx_v, rows_v, sem):
        wid = lax.axis_index("s") * SC_NC + lax.axis_index("c")
        base = wid * per_tile

        def step(g, carry):
            row0 = base + g * G_CHUNK
            pltpu.sync_copy(
                idx_hbm.at[pl.ds(pl.multiple_of(row0 // G_ROWS, G_K), G_K)],
                idx_v)
            handles = [
                pltpu.async_copy(
                    table_hbm.at[idx_v.at[j]],
                    rows_v.at[pl.ds(j * G_ROWS, G_ROWS)], sem)
                for j in range(G_K)
            ]
            for h in handles:
                h.wait()
            pltpu.sync_copy(rows_v, out_hbm.at[pl.ds(row0, G_CHUNK)])
            return carry

        lax.fori_loop(0, steps_per_tile, step, 0)

    return gk(table, idx2)


R_HALF = 25000
R_SP = 25008
S_DUMP = R_HALF
S_ZROWS = R_SP // SC_NS
A_K = 2
A_CHUNK = A_K * G_ROWS


def _sc_scatter(msgs, idx_loc2, zeros, steps_per_tile):
    e_pad = msgs.shape[0]
    assert e_pad == steps_per_tile * A_CHUNK * SC_NS
    core_idx_rows = e_pad // G_ROWS
    mesh = plsc.VectorSubcoreMesh(core_axis_name="c", subcore_axis_name="s")

    @functools.partial(
        pl.kernel, mesh=mesh,
        out_type=jax.ShapeDtypeStruct((2 * R_SP, FLAT), jnp.float32),
        scratch_types=[
            pltpu.VMEM((A_K, G_ROWS), jnp.int32),
            pltpu.VMEM((A_CHUNK, FLAT), jnp.float32),
            pltpu.VMEM_SHARED((R_SP, FLAT), jnp.float32),
            pltpu.SemaphoreType.DMA,
        ],
        compiler_params=pltpu.CompilerParams(use_tc_tiling_on_sc=False),
    )
    def sk(msg_hbm, idx_hbm, zero_hbm, out_hbm, idx_v, msg_v, shared, sem):
        c = lax.axis_index("c")
        s = lax.axis_index("s")

        pltpu.sync_copy(zero_hbm.at[pl.ds(s * S_ZROWS, S_ZROWS)],
                        shared.at[pl.ds(s * S_ZROWS, S_ZROWS)])
        plsc.subcore_barrier()

        def step(g, carry):
            row0 = s * steps_per_tile * A_CHUNK + g * A_CHUNK
            irow = c * core_idx_rows + row0 // G_ROWS
            pltpu.sync_copy(
                idx_hbm.at[pl.ds(pl.multiple_of(irow, A_K), A_K)], idx_v)
            pltpu.sync_copy(msg_hbm.at[pl.ds(row0, A_CHUNK)], msg_v)
            handles = [
                pltpu.async_copy(msg_v.at[pl.ds(j * G_ROWS, G_ROWS)],
                                 shared.at[idx_v.at[j]], sem, add=True)
                for j in range(A_K)
            ]
            for h in handles:
                h.wait()
            return carry

        lax.fori_loop(0, steps_per_tile, step, 0)
        plsc.subcore_barrier()

        pltpu.sync_copy(shared.at[pl.ds(s * S_ZROWS, S_ZROWS)],
                        out_hbm.at[pl.ds(c * R_SP + s * S_ZROWS, S_ZROWS)])

    return sk(msgs, idx_loc2, zeros)


CW = 8


def _sc_count(idx_loc2, ones, zeros, steps_per_tile):
    core_idx_rows = idx_loc2.shape[0] // 2
    assert core_idx_rows * G_ROWS == steps_per_tile * G_CHUNK * SC_NS
    mesh = plsc.VectorSubcoreMesh(core_axis_name="c", subcore_axis_name="s")

    @functools.partial(
        pl.kernel, mesh=mesh,
        out_type=jax.ShapeDtypeStruct((2 * R_SP, CW), jnp.float32),
        scratch_types=[
            pltpu.VMEM((G_K, G_ROWS), jnp.int32),
            pltpu.VMEM((G_ROWS, CW), jnp.float32),
            pltpu.VMEM_SHARED((R_SP, CW), jnp.float32),
            pltpu.SemaphoreType.DMA,
        ],
        compiler_params=pltpu.CompilerParams(use_tc_tiling_on_sc=False),
    )
    def ck(idx_hbm, ones_hbm, zero_hbm, out_hbm, idx_v, ones_v, shared, sem):
        c = lax.axis_index("c")
        s = lax.axis_index("s")

        pltpu.sync_copy(ones_hbm, ones_v)
        pltpu.sync_copy(zero_hbm.at[pl.ds(s * S_ZROWS, S_ZROWS)],
                        shared.at[pl.ds(s * S_ZROWS, S_ZROWS)])
        plsc.subcore_barrier()

        def step(g, carry):
            irow = (c * core_idx_rows
                    + (s * steps_per_tile * G_CHUNK + g * G_CHUNK) // G_ROWS)
            pltpu.sync_copy(
                idx_hbm.at[pl.ds(pl.multiple_of(irow, G_K), G_K)], idx_v)
            handles = [
                pltpu.async_copy(ones_v, shared.at[idx_v.at[j]], sem, add=True)
                for j in range(G_K)
            ]
            for h in handles:
                h.wait()
            return carry

        lax.fori_loop(0, steps_per_tile, step, 0)
        plsc.subcore_barrier()

        pltpu.sync_copy(shared.at[pl.ds(s * S_ZROWS, S_ZROWS)],
                        out_hbm.at[pl.ds(c * R_SP + s * S_ZROWS, S_ZROWS)])

    return ck(idx_loc2, ones, zeros)


def _mlp_body(ps_ref, pr_ref, ex_ref, w1s_ref, w1r_ref, w1e_ref, b1_ref,
              w2_ref, b2_ref, w3_ref, b3_ref, out_ref):
    ps = ps_ref[...]
    pr = pr_ref[...]
    ex = ex_ref[...]
    x = (jnp.dot(ps, w1s_ref[...], preferred_element_type=jnp.float32)
         + jnp.dot(pr, w1r_ref[...], preferred_element_type=jnp.float32)
         + jnp.dot(ex, w1e_ref[...], preferred_element_type=jnp.float32)
         + b1_ref[...])
    x = x * jax.nn.sigmoid(x)
    x = jnp.dot(x, w2_ref[...], preferred_element_type=jnp.float32) + b2_ref[...]
    x = x * jax.nn.sigmoid(x)
    out_ref[...] = (jnp.dot(x, w3_ref[...], preferred_element_type=jnp.float32)
                    + b3_ref[...])


def _run_mlp(gath, ex, w1s, w1r, w1e, b1, w2, b2, w3, b3, num_edges):
    eb = 2000 if num_edges % 2000 == 0 else num_edges
    grid = num_edges // eb
    off = num_edges // eb
    wspec = lambda shape: pl.BlockSpec(shape, lambda e: (0, 0))
    return pl.pallas_call(
        _mlp_body,
        grid=(grid,),
        in_specs=[
            pl.BlockSpec((eb, F), lambda e: (e, 0)),
            pl.BlockSpec((eb, F), lambda e: (e + off, 0)),
            pl.BlockSpec((eb, 8), lambda e: (e, 0)),
            wspec((F, HID)), wspec((F, HID)), wspec((8, HID)), wspec((1, HID)),
            wspec((HID, HID)), wspec((1, HID)),
            wspec((HID, FLAT)), wspec((1, FLAT)),
        ],
        out_specs=pl.BlockSpec((eb, FLAT), lambda e: (e, 0)),
        out_shape=jax.ShapeDtypeStruct((num_edges, FLAT), jnp.float32),
    )(gath, gath, ex, w1s, w1r, w1e, b1, w2, b2, w3, b3)


def _t64(m):
    rows = [m[j * 8 + i:j * 8 + i + 1, :] for i in range(8) for j in range(8)]
    return jnp.concatenate(rows, axis=0)


def _bmm(a, b):
    outs = []
    for i in range(8):
        acc = a[i * 8:i * 8 + 1, :] * b[0:8, :]
        for k in range(1, 8):
            acc = acc + a[i * 8 + k:i * 8 + k + 1, :] * b[k * 8:k * 8 + 8, :]
        outs.append(acc)
    return jnp.concatenate(outs, axis=0)


def _expm_body(agg_ref, h_ref, out_ref):
    agg = agg_ref[...]
    cnt = jnp.maximum(agg[FLAT:FLAT + 1, :], 1.0)
    mean = agg[0:FLAT, :] / cnt
    abs_sum = jnp.sum(jnp.abs(mean), axis=0, keepdims=True)
    r = lax.broadcasted_iota(jnp.int32, (FLAT, 1), 0)
    eye = ((r // 8) == (r % 8)).astype(jnp.float32)
    m = jnp.where(abs_sum == 0.0, eye, mean)
    skew = 0.5 * (m - _t64(m))
    a = skew * (1.0 / 256.0)
    b = m.shape[1]
    eye_b = jnp.broadcast_to(eye, (FLAT, b))
    term = eye_b
    result = eye_b
    for k in range(1, 13):
        term = _bmm(term, a) * (1.0 / k)
        result = result + term
    for _ in range(8):
        result = _bmm(result, result)
    u = result
    h = h_ref[...]
    out_ref[...] = _bmm(u, _bmm(h, _t64(u)))


def _run_expm(agg_t, h_t, npad):
    nb = 2048 if npad % 2048 == 0 else npad
    grid = npad // nb
    return pl.pallas_call(
        _expm_body,
        grid=(grid,),
        in_specs=[
            pl.BlockSpec((F, nb), lambda i: (0, i)),
            pl.BlockSpec((FLAT, nb), lambda i: (0, i)),
        ],
        out_specs=pl.BlockSpec((FLAT, nb), lambda i: (0, i)),
        out_shape=jax.ShapeDtypeStruct((FLAT, npad), jnp.float32),
    )(agg_t, h_t)


def kernel(h_i, edge_index, node_params, edge_params, W1, b1, W2, b2, W3, b3):
    n = h_i.shape[0]
    num_edges = edge_index.shape[1]
    sender = edge_index[0]
    receiver = edge_index[1]

    h_flat = h_i.reshape(n, FLAT)
    p = jnp.concatenate(
        [h_flat, node_params, edge_params,
         jnp.zeros((n, F - FLAT - 3), jnp.float32)], axis=1)

    idx_all = jnp.concatenate([sender, receiver])
    steps = -(-2 * num_edges // (G_CHUNK * SC_NW))
    b_pad = steps * G_CHUNK * SC_NW
    idx_pad = jnp.pad(idx_all, (0, b_pad - 2 * num_edges))
    gath = _sc_gather(p, idx_pad, steps)

    eps = gath[:num_edges, 66:67]
    epr = gath[num_edges:2 * num_edges, 66:67]
    scram = jnp.concatenate([eps, epr], axis=0).reshape(num_edges, 2)
    ex = jnp.concatenate([scram, jnp.zeros((num_edges, 6), jnp.float32)], axis=1)

    zcol = jnp.zeros((F - 66, HID), jnp.float32)
    w1s = jnp.concatenate([W1[0:64], W1[128:130], zcol], axis=0)
    w1r = jnp.concatenate([W1[64:128], W1[130:132], zcol], axis=0)
    w1e = jnp.concatenate([W1[132:134], jnp.zeros((6, HID), jnp.float32)], axis=0)

    m80 = _run_mlp(gath, ex, w1s, w1r, w1e, b1.reshape(1, HID),
                   W2, b2.reshape(1, HID), W3, b3.reshape(1, FLAT), num_edges)

    c_steps = -(-num_edges // (G_CHUNK * SC_NS))
    e_pad = c_steps * G_CHUNK * SC_NS
    a_steps = e_pad // (A_CHUNK * SC_NS)
    rpad = jnp.pad(receiver, (0, e_pad - num_edges), constant_values=-1)
    loc0 = jnp.where((rpad >= 0) & (rpad < R_HALF), rpad, S_DUMP)
    r1 = rpad - R_HALF
    loc1 = jnp.where((r1 >= 0) & (r1 < R_HALF), r1, S_DUMP)
    idx_loc2 = jnp.concatenate([loc0, loc1]).reshape(2 * e_pad // 128, 128)
    msgs = jnp.pad(m80, ((0, e_pad - num_edges), (0, 0)))

    agg2 = _sc_scatter(msgs, idx_loc2,
                       jnp.zeros((R_SP, FLAT), jnp.float32), a_steps)
    cnt2 = _sc_count(idx_loc2, jnp.ones((G_ROWS, CW), jnp.float32),
                     jnp.zeros((R_SP, CW), jnp.float32), c_steps)
    aggm = jnp.concatenate(
        [agg2[:R_HALF], agg2[R_SP:R_SP + R_HALF]], axis=0)
    cnt = jnp.concatenate(
        [cnt2[:R_HALF, :1], cnt2[R_SP:R_SP + R_HALF, :1]], axis=0)
    agg80 = jnp.concatenate(
        [aggm, cnt, jnp.zeros((n, F - FLAT - 1), jnp.float32)], axis=1)

    nb = 2048
    npad = ((n + nb - 1) // nb) * nb
    agg_t = jnp.pad(agg80, ((0, npad - n), (0, 0))).T
    h_t = jnp.pad(h_flat, ((0, npad - n), (0, 0))).T

    out_t = _run_expm(agg_t, h_t, npad)
    return out_t.T[:n].reshape(n, D, D)
